# Initial kernel scaffold; baseline (speedup 1.0000x reference)
import jax
import jax.numpy as jnp
from jax.experimental import pallas as pl


def kernel(z, x, cu_seqlens, table, W1, b1, W2, b2, gamma, beta):
    raise NotImplementedError("write your pallas kernel here")


# speedup vs baseline: 3.5856x; 3.5856x over previous
import functools

import jax
import jax.numpy as jnp
from jax.experimental import pallas as pl
from jax.experimental.pallas import tpu as pltpu

_TOK = 32768
_D = 128
_T = 2048


def _body(z_ref, x_ref, table_ref, W1_ref, b1_ref, W2_ref, b2_ref,
          g_ref, bt_ref, out_ref):
    x = x_ref[...]
    w1 = W1_ref[...]
    p = (x[:, 0:1] * w1[0:1, :]
         + x[:, 1:2] * w1[1:2, :]
         + x[:, 2:3] * w1[2:3, :]
         + b1_ref[...])
    p = p * jax.nn.sigmoid(p)
    h = jnp.dot(p, W2_ref[...], preferred_element_type=jnp.float32)
    h = h + b2_ref[...]

    z = z_ref[...]
    iota = jax.lax.broadcasted_iota(jnp.int32, (_T, _D), 1)
    oh = (z == iota).astype(jnp.float32)
    h = h + jnp.dot(oh, table_ref[...], preferred_element_type=jnp.float32)

    mean = jnp.mean(h, axis=1, keepdims=True)
    c = h - mean
    var = jnp.mean(c * c, axis=1, keepdims=True)
    out_ref[...] = c * jax.lax.rsqrt(var + 1e-5) * g_ref[...] + bt_ref[...]


@jax.jit
def _fused(z2, x, table_pad, W1, b1, W2, b2, gamma, beta):
    grid = (_TOK // _T,)
    return pl.pallas_call(
        _body,
        grid=grid,
        in_specs=[
            pl.BlockSpec((_T, 1), lambda i: (i, 0)),
            pl.BlockSpec((_T, 3), lambda i: (i, 0)),
            pl.BlockSpec((_D, _D), lambda i: (0, 0)),
            pl.BlockSpec((3, _D), lambda i: (0, 0)),
            pl.BlockSpec((1, _D), lambda i: (0, 0)),
            pl.BlockSpec((_D, _D), lambda i: (0, 0)),
            pl.BlockSpec((1, _D), lambda i: (0, 0)),
            pl.BlockSpec((1, _D), lambda i: (0, 0)),
            pl.BlockSpec((1, _D), lambda i: (0, 0)),
        ],
        out_specs=pl.BlockSpec((_T, _D), lambda i: (i, 0)),
        out_shape=jax.ShapeDtypeStruct((_TOK, _D), jnp.float32),
        compiler_params=pltpu.CompilerParams(
            dimension_semantics=("arbitrary",),
        ),
    )(z2, x, table_pad, W1, b1, W2, b2, gamma, beta)


def kernel(z, x, cu_seqlens, table, W1, b1, W2, b2, gamma, beta):
    del cu_seqlens
    z2 = z.astype(jnp.int32).reshape(_TOK, 1)
    table_pad = jnp.zeros((_D, _D), jnp.float32).at[:table.shape[0]].set(table)
    return _fused(z2, x, table_pad, W1,
                  b1.reshape(1, _D), W2, b2.reshape(1, _D),
                  gamma.reshape(1, _D), beta.reshape(1, _D))
